# Initial kernel scaffold; baseline (speedup 1.0000x reference)
import functools

import jax
import jax.numpy as jnp
from jax import lax
from jax.experimental import pallas as pl
from jax.experimental.pallas import tpu as pltpu
from jax.experimental.pallas import tpu_sc as plsc

_NC = 2
_NS = 16
_NW = _NC * _NS
_LANES = 16
_CH = 128
_ZR = 16
_G = 64
_BM = 1024


def _sc_segment_gather_add(h_pad, src_t, dst_t, n_pad, nch):
    d = h_pad.shape[1]
    rows_per_tile = n_pad // _NS
    mesh = plsc.VectorSubcoreMesh(core_axis_name="c", subcore_axis_name="s")

    @functools.partial(
        pl.kernel,
        mesh=mesh,
        out_type=jax.ShapeDtypeStruct((_NC, n_pad, d), jnp.float32),
        scratch_types=[
            pltpu.VMEM((nch, _CH), jnp.int32),
            pltpu.VMEM((nch, _CH), jnp.int32),
            pltpu.VMEM((_CH, d), jnp.float32),
            pltpu.VMEM((_ZR, d), jnp.float32),
            pltpu.VMEM_SHARED((n_pad, d), jnp.float32),
            pltpu.SemaphoreType.DMA,
        ],
    )
    def k(h_hbm, src_hbm, dst_hbm, out_hbm, srcv, dstv, rows, zbuf, acc, sem):
        cid = lax.axis_index("c")
        sid = lax.axis_index("s")
        wid = sid * _NC + cid

        zv = jnp.zeros((_LANES,), jnp.float32)

        @pl.loop(0, _ZR)
        def _(r):
            @pl.loop(0, d, step=_LANES)
            def _(j):
                zbuf[r, pl.ds(j, _LANES)] = zv

        base = sid * rows_per_tile

        @pl.loop(0, rows_per_tile, step=_ZR)
        def _(r):
            pltpu.sync_copy(zbuf, acc.at[pl.ds(base + r, _ZR)])

        pltpu.sync_copy(src_hbm.at[wid], srcv)
        pltpu.sync_copy(dst_hbm.at[wid], dstv)
        plsc.subcore_barrier()

        @pl.loop(0, nch)
        def _(c):
            pltpu.async_copy(h_hbm.at[srcv.at[c]], rows, sem).wait()
            pltpu.sync_copy(rows, acc.at[dstv.at[c]], add=True)

        plsc.subcore_barrier()
        pltpu.sync_copy(
            acc.at[pl.ds(base, rows_per_tile)],
            out_hbm.at[cid].at[pl.ds(base, rows_per_tile)],
        )

    return k(h_pad, src_t, dst_t)


def _tc_layer(p0, p1, h, w, n_pad):
    d = h.shape[1]

    def body(p0_ref, p1_ref, h_ref, w_ref, o_ref):
        s = p0_ref[...] + p1_ref[...]
        o_ref[...] = (
            jnp.maximum(
                jnp.dot(
                    s,
                    w_ref[...],
                    preferred_element_type=jnp.float32,
                    precision=lax.Precision.HIGHEST,
                ),
                0.0,
            )
            + h_ref[...]
        )

    return pl.pallas_call(
        body,
        grid=(n_pad // _BM,),
        in_specs=[
            pl.BlockSpec((_BM, d), lambda i: (i, 0)),
            pl.BlockSpec((_BM, d), lambda i: (i, 0)),
            pl.BlockSpec((_BM, d), lambda i: (i, 0)),
            pl.BlockSpec((d, d), lambda i: (0, 0)),
        ],
        out_specs=pl.BlockSpec((_BM, d), lambda i: (i, 0)),
        out_shape=jax.ShapeDtypeStruct((n_pad, d), jnp.float32),
    )(p0, p1, h, w)


def _tc_layer2_pool(p0, p1, h, w, w_out, batch2d, n_pad, t_std, t_mean):
    d = h.shape[1]
    nsteps = n_pad // _BM

    def body(p0_ref, p1_ref, h_ref, w_ref, wo_ref, b_ref, o_ref):
        i = pl.program_id(0)
        s = p0_ref[...] + p1_ref[...]
        h2 = (
            jnp.maximum(
                jnp.dot(
                    s,
                    w_ref[...],
                    preferred_element_type=jnp.float32,
                    precision=lax.Precision.HIGHEST,
                ),
                0.0,
            )
            + h_ref[...]
        )
        y = jnp.dot(
            h2,
            wo_ref[...],
            preferred_element_type=jnp.float32,
            precision=lax.Precision.HIGHEST,
        )
        b = b_ref[...]
        gids = lax.broadcasted_iota(jnp.int32, (_G, _BM), 0)
        mask = b == gids
        contrib = jnp.sum(jnp.where(mask, y[:, 0][None, :], 0.0), axis=1)
        prev = jnp.where(i == 0, jnp.zeros((1, _G), jnp.float32), o_ref[...])
        tot = prev + contrib[None, :]
        o_ref[...] = jnp.where(i == nsteps - 1, tot * t_std + t_mean, tot)

    return pl.pallas_call(
        body,
        grid=(nsteps,),
        in_specs=[
            pl.BlockSpec((_BM, d), lambda i: (i, 0)),
            pl.BlockSpec((_BM, d), lambda i: (i, 0)),
            pl.BlockSpec((_BM, d), lambda i: (i, 0)),
            pl.BlockSpec((d, d), lambda i: (0, 0)),
            pl.BlockSpec((d, 1), lambda i: (0, 0)),
            pl.BlockSpec((1, _BM), lambda i: (0, i)),
        ],
        out_specs=pl.BlockSpec((1, _G), lambda i: (0, 0)),
        out_shape=jax.ShapeDtypeStruct((1, _G), jnp.float32),
    )(p0, p1, h, w, w_out, batch2d)


def kernel(x, edge_index, batch, W1, W2, W_out):
    n, d = x.shape
    e = edge_index.shape[1]

    n_pad = ((n + 1 + _BM - 1) // _BM) * _BM
    nch = -(-e // (_NW * _CH))
    cap = _NW * nch * _CH

    src = edge_index[0].astype(jnp.int32)
    dst = edge_index[1].astype(jnp.int32)
    src_t = jnp.full((cap,), n, jnp.int32).at[:e].set(src).reshape(_NW, nch, _CH)
    dst_t = jnp.full((cap,), n, jnp.int32).at[:e].set(dst).reshape(_NW, nch, _CH)
    batch2d = jnp.concatenate(
        [batch.astype(jnp.int32), jnp.full((n_pad - n,), _G, jnp.int32)]
    ).reshape(1, n_pad)

    h0 = jnp.concatenate([x, jnp.zeros((n_pad - n, d), x.dtype)], axis=0)

    p = _sc_segment_gather_add(h0, src_t, dst_t, n_pad, nch)
    h1 = _tc_layer(p[0], p[1], h0, W1, n_pad)
    q = _sc_segment_gather_add(h1, src_t, dst_t, n_pad, nch)
    out = _tc_layer2_pool(
        q[0], q[1], h1, W2, W_out, batch2d, n_pad, 25.0, 500.0
    )
    return out.reshape(_G)


# speedup vs baseline: 4.6698x; 4.6698x over previous
import functools

import jax
import jax.numpy as jnp
from jax import lax
from jax.experimental import pallas as pl
from jax.experimental.pallas import tpu as pltpu
from jax.experimental.pallas import tpu_sc as plsc

_NC = 2
_NS = 16
_NW = _NC * _NS
_LANES = 16
_CH = 128
_ZR = 16
_G = 64
_BM = 1024


def _sc_segment_gather_add(h_pad, src_t, dst_t, n_pad, nch):
    d = h_pad.shape[1]
    rows_per_tile = n_pad // _NS
    mesh = plsc.VectorSubcoreMesh(core_axis_name="c", subcore_axis_name="s")

    @functools.partial(
        pl.kernel,
        mesh=mesh,
        out_type=jax.ShapeDtypeStruct((_NC, n_pad, d), jnp.float32),
        scratch_types=[
            pltpu.VMEM((nch, _CH), jnp.int32),
            pltpu.VMEM((nch, _CH), jnp.int32),
            pltpu.VMEM((_CH, d), jnp.float32),
            pltpu.VMEM((_ZR, d), jnp.float32),
            pltpu.VMEM_SHARED((n_pad, d), jnp.float32),
            pltpu.SemaphoreType.DMA,
        ],
    )
    def k(h_hbm, src_hbm, dst_hbm, out_hbm, srcv, dstv, rows, zbuf, acc, sem):
        cid = lax.axis_index("c")
        sid = lax.axis_index("s")
        wid = sid * _NC + cid

        zv = jnp.zeros((_LANES,), jnp.float32)

        @pl.loop(0, _ZR)
        def _(r):
            @pl.loop(0, d, step=_LANES)
            def _(j):
                zbuf[r, pl.ds(j, _LANES)] = zv

        base = sid * rows_per_tile

        @pl.loop(0, rows_per_tile, step=_ZR)
        def _(r):
            pltpu.sync_copy(zbuf, acc.at[pl.ds(base + r, _ZR)])

        pltpu.sync_copy(src_hbm.at[wid], srcv)
        pltpu.sync_copy(dst_hbm.at[wid], dstv)
        plsc.subcore_barrier()

        @pl.loop(0, nch)
        def _(c):
            pltpu.async_copy(h_hbm.at[srcv.at[c]], rows, sem).wait()
            pltpu.sync_copy(rows, acc.at[dstv.at[c]], add=True)

        plsc.subcore_barrier()
        pltpu.sync_copy(
            acc.at[pl.ds(base, rows_per_tile)],
            out_hbm.at[cid].at[pl.ds(base, rows_per_tile)],
        )

    return k(h_pad, src_t, dst_t)


def _round_bf16(a):
    return a.astype(jnp.bfloat16).astype(jnp.float32)


def _tc_layer(p0, p1, h, w, n_pad):
    d = h.shape[1]

    def body(p0_ref, p1_ref, h_ref, w_ref, o_ref, or_ref):
        s = p0_ref[...] + p1_ref[...]
        v = (
            jnp.maximum(
                jnp.dot(
                    s,
                    w_ref[...],
                    preferred_element_type=jnp.float32,
                    precision=lax.Precision.HIGHEST,
                ),
                0.0,
            )
            + h_ref[...]
        )
        o_ref[...] = v
        or_ref[...] = _round_bf16(v)

    return pl.pallas_call(
        body,
        grid=(n_pad // _BM,),
        in_specs=[
            pl.BlockSpec((_BM, d), lambda i: (i, 0)),
            pl.BlockSpec((_BM, d), lambda i: (i, 0)),
            pl.BlockSpec((_BM, d), lambda i: (i, 0)),
            pl.BlockSpec((d, d), lambda i: (0, 0)),
        ],
        out_specs=[
            pl.BlockSpec((_BM, d), lambda i: (i, 0)),
            pl.BlockSpec((_BM, d), lambda i: (i, 0)),
        ],
        out_shape=[
            jax.ShapeDtypeStruct((n_pad, d), jnp.float32),
            jax.ShapeDtypeStruct((n_pad, d), jnp.float32),
        ],
    )(p0, p1, h, w)


def _tc_layer2_pool(p0, p1, h, w, w_out, batch2d, n_pad, t_std, t_mean):
    d = h.shape[1]
    nsteps = n_pad // _BM

    def body(p0_ref, p1_ref, h_ref, w_ref, wo_ref, b_ref, o_ref, pool_ref):
        i = pl.program_id(0)
        s = p0_ref[...] + p1_ref[...]
        h2 = (
            jnp.maximum(
                jnp.dot(
                    s,
                    w_ref[...],
                    preferred_element_type=jnp.float32,
                    precision=lax.Precision.HIGHEST,
                ),
                0.0,
            )
            + h_ref[...]
        )
        b = b_ref[...]
        gids = lax.broadcasted_iota(jnp.int32, (_G, _BM), 0)
        mask = (b == gids).astype(jnp.float32)
        contrib = jnp.dot(
            mask,
            h2,
            preferred_element_type=jnp.float32,
            precision=lax.Precision.HIGHEST,
        )

        @pl.when(i == 0)
        def _():
            pool_ref[...] = jnp.zeros((_G, d), jnp.float32)

        pool_ref[...] += contrib

        @pl.when(i == nsteps - 1)
        def _():
            g = _round_bf16(pool_ref[...])
            y = jnp.dot(
                g,
                wo_ref[...],
                preferred_element_type=jnp.float32,
                precision=lax.Precision.HIGHEST,
            )
            o_ref[...] = y[:, 0][None, :] * t_std + t_mean

    return pl.pallas_call(
        body,
        grid=(nsteps,),
        in_specs=[
            pl.BlockSpec((_BM, d), lambda i: (i, 0)),
            pl.BlockSpec((_BM, d), lambda i: (i, 0)),
            pl.BlockSpec((_BM, d), lambda i: (i, 0)),
            pl.BlockSpec((d, d), lambda i: (0, 0)),
            pl.BlockSpec((d, 1), lambda i: (0, 0)),
            pl.BlockSpec((1, _BM), lambda i: (0, i)),
        ],
        out_specs=pl.BlockSpec((1, _G), lambda i: (0, 0)),
        out_shape=jax.ShapeDtypeStruct((1, _G), jnp.float32),
        scratch_shapes=[pltpu.VMEM((_G, d), jnp.float32)],
    )(p0, p1, h, w, w_out, batch2d)


def kernel(x, edge_index, batch, W1, W2, W_out):
    n, d = x.shape
    e = edge_index.shape[1]

    n_pad = ((n + 1 + _BM - 1) // _BM) * _BM
    nch = -(-e // (_NW * _CH))
    cap = _NW * nch * _CH

    src = edge_index[0].astype(jnp.int32)
    dst = edge_index[1].astype(jnp.int32)
    src_t = jnp.full((cap,), n, jnp.int32).at[:e].set(src).reshape(_NW, nch, _CH)
    dst_t = jnp.full((cap,), n, jnp.int32).at[:e].set(dst).reshape(_NW, nch, _CH)
    batch2d = jnp.concatenate(
        [batch.astype(jnp.int32), jnp.full((n_pad - n,), _G, jnp.int32)]
    ).reshape(1, n_pad)

    zpad = jnp.zeros((n_pad - n, d), jnp.float32)
    h0 = jnp.concatenate([x, zpad], axis=0)
    h0r = jnp.concatenate([_round_bf16(x), zpad], axis=0)
    w1r = _round_bf16(W1)
    w2r = _round_bf16(W2)
    wor = _round_bf16(W_out)

    p = _sc_segment_gather_add(h0r, src_t, dst_t, n_pad, nch)
    h1, h1r = _tc_layer(p[0], p[1], h0, w1r, n_pad)
    q = _sc_segment_gather_add(h1r, src_t, dst_t, n_pad, nch)
    out = _tc_layer2_pool(
        q[0], q[1], h1, w2r, wor, batch2d, n_pad, 25.0, 500.0
    )
    return out.reshape(_G)


# speedup vs baseline: 9.6214x; 2.0603x over previous
import functools

import jax
import jax.numpy as jnp
from jax import lax
from jax.experimental import pallas as pl
from jax.experimental.pallas import tpu as pltpu
from jax.experimental.pallas import tpu_sc as plsc

_NC = 2
_NS = 16
_NW = _NC * _NS
_LANES = 16
_CH = 128
_ZR = 16
_G = 64
_BM = 1024


def _sc_segment_gather_add(h_pad, src_t, dst_t, n_pad, nch):
    d = h_pad.shape[1]
    rows_per_tile = n_pad // _NS
    nch2 = nch // 2
    mesh = plsc.VectorSubcoreMesh(core_axis_name="c", subcore_axis_name="s")

    @functools.partial(
        pl.kernel,
        mesh=mesh,
        out_type=jax.ShapeDtypeStruct((_NC, n_pad, d), jnp.float32),
        scratch_types=[
            pltpu.VMEM((nch2, _CH), jnp.int32),
            pltpu.VMEM((nch2, _CH), jnp.int32),
            pltpu.VMEM((_CH, d), jnp.float32),
            pltpu.VMEM((_CH, d), jnp.float32),
            pltpu.VMEM((_ZR, d), jnp.float32),
            pltpu.VMEM_SHARED((n_pad, d), jnp.float32),
            pltpu.SemaphoreType.DMA,
            pltpu.SemaphoreType.DMA,
            pltpu.SemaphoreType.DMA,
            pltpu.SemaphoreType.DMA,
        ],
    )
    def k(h_hbm, src_hbm, dst_hbm, out_hbm, srcv, dstv, rows_a, rows_b,
          zbuf, acc, gsem_a, gsem_b, ssem_a, ssem_b):
        cid = lax.axis_index("c")
        sid = lax.axis_index("s")
        wid = sid * _NC + cid

        zv = jnp.zeros((_LANES,), jnp.float32)

        @pl.loop(0, _ZR)
        def _(r):
            @pl.loop(0, d, step=_LANES)
            def _(j):
                zbuf[r, pl.ds(j, _LANES)] = zv

        base = sid * rows_per_tile

        @pl.loop(0, rows_per_tile, step=_ZR)
        def _(r):
            pltpu.sync_copy(zbuf, acc.at[pl.ds(base + r, _ZR)])

        plsc.subcore_barrier()

        for phase in range(2):
            pltpu.sync_copy(src_hbm.at[wid].at[pl.ds(phase * nch2, nch2)], srcv)
            pltpu.sync_copy(dst_hbm.at[wid].at[pl.ds(phase * nch2, nch2)], dstv)

            pltpu.async_copy(h_hbm.at[srcv.at[0]], rows_a, gsem_a)
            pltpu.async_copy(h_hbm.at[srcv.at[1]], rows_b, gsem_b)

            @pl.loop(0, nch2 - 2, step=2)
            def _(c):
                pltpu.make_async_copy(h_hbm.at[srcv.at[c]], rows_a, gsem_a).wait()
                pltpu.async_copy(rows_a, acc.at[dstv.at[c]], ssem_a, add=True)
                pltpu.make_async_copy(h_hbm.at[srcv.at[c + 1]], rows_b, gsem_b).wait()
                pltpu.async_copy(rows_b, acc.at[dstv.at[c + 1]], ssem_b, add=True)
                pltpu.make_async_copy(rows_a, acc.at[dstv.at[c]], ssem_a).wait()
                pltpu.async_copy(h_hbm.at[srcv.at[c + 2]], rows_a, gsem_a)
                pltpu.make_async_copy(rows_b, acc.at[dstv.at[c + 1]], ssem_b).wait()
                pltpu.async_copy(h_hbm.at[srcv.at[c + 3]], rows_b, gsem_b)

            pltpu.make_async_copy(h_hbm.at[srcv.at[nch2 - 2]], rows_a, gsem_a).wait()
            pltpu.sync_copy(rows_a, acc.at[dstv.at[nch2 - 2]], add=True)
            pltpu.make_async_copy(h_hbm.at[srcv.at[nch2 - 1]], rows_b, gsem_b).wait()
            pltpu.sync_copy(rows_b, acc.at[dstv.at[nch2 - 1]], add=True)

        plsc.subcore_barrier()
        pltpu.sync_copy(
            acc.at[pl.ds(base, rows_per_tile)],
            out_hbm.at[cid].at[pl.ds(base, rows_per_tile)],
        )

    return k(h_pad, src_t, dst_t)


def _round_bf16(a):
    return a.astype(jnp.bfloat16).astype(jnp.float32)


def _tc_layer(p0, p1, h, w, n_pad):
    d = h.shape[1]

    def body(p0_ref, p1_ref, h_ref, w_ref, o_ref, or_ref):
        s = p0_ref[...] + p1_ref[...]
        v = (
            jnp.maximum(
                jnp.dot(
                    s,
                    w_ref[...],
                    preferred_element_type=jnp.float32,
                    precision=lax.Precision.HIGHEST,
                ),
                0.0,
            )
            + h_ref[...]
        )
        o_ref[...] = v
        or_ref[...] = _round_bf16(v)

    return pl.pallas_call(
        body,
        grid=(n_pad // _BM,),
        in_specs=[
            pl.BlockSpec((_BM, d), lambda i: (i, 0)),
            pl.BlockSpec((_BM, d), lambda i: (i, 0)),
            pl.BlockSpec((_BM, d), lambda i: (i, 0)),
            pl.BlockSpec((d, d), lambda i: (0, 0)),
        ],
        out_specs=[
            pl.BlockSpec((_BM, d), lambda i: (i, 0)),
            pl.BlockSpec((_BM, d), lambda i: (i, 0)),
        ],
        out_shape=[
            jax.ShapeDtypeStruct((n_pad, d), jnp.float32),
            jax.ShapeDtypeStruct((n_pad, d), jnp.float32),
        ],
    )(p0, p1, h, w)


def _tc_layer2_pool(p0, p1, h, w, w_out, batch2d, n_pad, t_std, t_mean):
    d = h.shape[1]
    nsteps = n_pad // _BM

    def body(p0_ref, p1_ref, h_ref, w_ref, wo_ref, b_ref, o_ref, pool_ref):
        i = pl.program_id(0)
        s = p0_ref[...] + p1_ref[...]
        h2 = (
            jnp.maximum(
                jnp.dot(
                    s,
                    w_ref[...],
                    preferred_element_type=jnp.float32,
                    precision=lax.Precision.HIGHEST,
                ),
                0.0,
            )
            + h_ref[...]
        )
        b = b_ref[...]
        gids = lax.broadcasted_iota(jnp.int32, (_G, _BM), 0)
        mask = (b == gids).astype(jnp.float32)
        contrib = jnp.dot(
            mask,
            h2,
            preferred_element_type=jnp.float32,
            precision=lax.Precision.HIGHEST,
        )

        @pl.when(i == 0)
        def _():
            pool_ref[...] = jnp.zeros((_G, d), jnp.float32)

        pool_ref[...] += contrib

        @pl.when(i == nsteps - 1)
        def _():
            g = _round_bf16(pool_ref[...])
            y = jnp.dot(
                g,
                wo_ref[...],
                preferred_element_type=jnp.float32,
                precision=lax.Precision.HIGHEST,
            )
            o_ref[...] = y[:, 0][None, :] * t_std + t_mean

    return pl.pallas_call(
        body,
        grid=(nsteps,),
        in_specs=[
            pl.BlockSpec((_BM, d), lambda i: (i, 0)),
            pl.BlockSpec((_BM, d), lambda i: (i, 0)),
            pl.BlockSpec((_BM, d), lambda i: (i, 0)),
            pl.BlockSpec((d, d), lambda i: (0, 0)),
            pl.BlockSpec((d, 1), lambda i: (0, 0)),
            pl.BlockSpec((1, _BM), lambda i: (0, i)),
        ],
        out_specs=pl.BlockSpec((1, _G), lambda i: (0, 0)),
        out_shape=jax.ShapeDtypeStruct((1, _G), jnp.float32),
        scratch_shapes=[pltpu.VMEM((_G, d), jnp.float32)],
    )(p0, p1, h, w, w_out, batch2d)


def kernel(x, edge_index, batch, W1, W2, W_out):
    n, d = x.shape
    e = edge_index.shape[1]

    n_pad = ((n + 1 + _BM - 1) // _BM) * _BM
    nch = -(-e // (_NW * _CH))
    nch = ((nch + 3) // 4) * 4
    cap = _NW * nch * _CH

    src = edge_index[0].astype(jnp.int32)
    dst = edge_index[1].astype(jnp.int32)
    pad_idx = n + (jnp.arange(cap - e, dtype=jnp.int32) % (n_pad - n))
    src_t = jnp.concatenate([src, pad_idx]).reshape(_NW, nch, _CH)
    dst_t = jnp.concatenate([dst, pad_idx]).reshape(_NW, nch, _CH)
    batch2d = jnp.concatenate(
        [batch.astype(jnp.int32), jnp.full((n_pad - n,), _G, jnp.int32)]
    ).reshape(1, n_pad)

    zpad = jnp.zeros((n_pad - n, d), jnp.float32)
    h0 = jnp.concatenate([x, zpad], axis=0)
    h0r = jnp.concatenate([_round_bf16(x), zpad], axis=0)
    w1r = _round_bf16(W1)
    w2r = _round_bf16(W2)
    wor = _round_bf16(W_out)

    p = _sc_segment_gather_add(h0r, src_t, dst_t, n_pad, nch)
    h1, h1r = _tc_layer(p[0], p[1], h0, w1r, n_pad)
    q = _sc_segment_gather_add(h1r, src_t, dst_t, n_pad, nch)
    out = _tc_layer2_pool(
        q[0], q[1], h1, w2r, wor, batch2d, n_pad, 25.0, 500.0
    )
    return out.reshape(_G)


# speedup vs baseline: 10.8394x; 1.1266x over previous
import functools

import jax
import jax.numpy as jnp
from jax import lax
from jax.experimental import pallas as pl
from jax.experimental.pallas import tpu as pltpu
from jax.experimental.pallas import tpu_sc as plsc

_NC = 2
_NS = 16
_NW = _NC * _NS
_LANES = 16
_CH = 64
_NB = 4
_NPH = 4
_ZR = 16
_G = 64
_BM = 1024


def _sc_segment_gather_add(h_pad, src_t, dst_t, n_pad, nch):
    d = h_pad.shape[1]
    rows_per_tile = n_pad // _NS
    nch2 = nch // _NPH
    mesh = plsc.VectorSubcoreMesh(core_axis_name="c", subcore_axis_name="s")

    @functools.partial(
        pl.kernel,
        mesh=mesh,
        out_type=jax.ShapeDtypeStruct((_NC, n_pad, d), jnp.float32),
        scratch_types=(
            [
                pltpu.VMEM((nch2, _CH), jnp.int32),
                pltpu.VMEM((nch2, _CH), jnp.int32),
            ]
            + [pltpu.VMEM((_CH, d), jnp.float32)] * _NB
            + [
                pltpu.VMEM((_ZR, d), jnp.float32),
                pltpu.VMEM_SHARED((n_pad, d), jnp.float32),
            ]
            + [pltpu.SemaphoreType.DMA] * (2 * _NB)
        ),
    )
    def k(h_hbm, src_hbm, dst_hbm, out_hbm, srcv, dstv, *rest):
        rows_bufs = rest[:_NB]
        zbuf = rest[_NB]
        acc = rest[_NB + 1]
        gsems = rest[_NB + 2 : 2 * _NB + 2]
        ssems = rest[2 * _NB + 2 :]
        cid = lax.axis_index("c")
        sid = lax.axis_index("s")
        wid = sid * _NC + cid

        zv = jnp.zeros((_LANES,), jnp.float32)

        @pl.loop(0, _ZR)
        def _(r):
            @pl.loop(0, d, step=_LANES)
            def _(j):
                zbuf[r, pl.ds(j, _LANES)] = zv

        base = sid * rows_per_tile

        @pl.loop(0, rows_per_tile, step=_ZR)
        def _(r):
            pltpu.sync_copy(zbuf, acc.at[pl.ds(base + r, _ZR)])

        plsc.subcore_barrier()

        nb = len(rows_bufs)
        for phase in range(_NPH):
            pltpu.sync_copy(src_hbm.at[wid].at[pl.ds(phase * nch2, nch2)], srcv)
            pltpu.sync_copy(dst_hbm.at[wid].at[pl.ds(phase * nch2, nch2)], dstv)

            for k in range(nb):
                pltpu.async_copy(h_hbm.at[srcv.at[k]], rows_bufs[k], gsems[k])

            @pl.loop(0, nch2 - nb, step=nb)
            def _(c):
                for k in range(nb):
                    pltpu.make_async_copy(
                        h_hbm.at[srcv.at[c + k]], rows_bufs[k], gsems[k]).wait()
                    pltpu.async_copy(
                        rows_bufs[k], acc.at[dstv.at[c + k]], ssems[k], add=True)
                for k in range(nb):
                    pltpu.make_async_copy(
                        rows_bufs[k], acc.at[dstv.at[c + k]], ssems[k]).wait()
                    pltpu.async_copy(
                        h_hbm.at[srcv.at[c + nb + k]], rows_bufs[k], gsems[k])

            for k in range(nb):
                pltpu.make_async_copy(
                    h_hbm.at[srcv.at[nch2 - nb + k]], rows_bufs[k], gsems[k]).wait()
                pltpu.sync_copy(rows_bufs[k], acc.at[dstv.at[nch2 - nb + k]], add=True)

        plsc.subcore_barrier()
        pltpu.sync_copy(
            acc.at[pl.ds(base, rows_per_tile)],
            out_hbm.at[cid].at[pl.ds(base, rows_per_tile)],
        )

    return k(h_pad, src_t, dst_t)


def _round_bf16(a):
    return a.astype(jnp.bfloat16).astype(jnp.float32)


def _tc_layer(p0, p1, h, w, n_pad):
    d = h.shape[1]

    def body(p0_ref, p1_ref, h_ref, w_ref, o_ref, or_ref):
        s = p0_ref[...] + p1_ref[...]
        v = (
            jnp.maximum(
                jnp.dot(
                    s,
                    w_ref[...],
                    preferred_element_type=jnp.float32,
                    precision=lax.Precision.HIGHEST,
                ),
                0.0,
            )
            + h_ref[...]
        )
        o_ref[...] = v
        or_ref[...] = _round_bf16(v)

    return pl.pallas_call(
        body,
        grid=(n_pad // _BM,),
        in_specs=[
            pl.BlockSpec((_BM, d), lambda i: (i, 0)),
            pl.BlockSpec((_BM, d), lambda i: (i, 0)),
            pl.BlockSpec((_BM, d), lambda i: (i, 0)),
            pl.BlockSpec((d, d), lambda i: (0, 0)),
        ],
        out_specs=[
            pl.BlockSpec((_BM, d), lambda i: (i, 0)),
            pl.BlockSpec((_BM, d), lambda i: (i, 0)),
        ],
        out_shape=[
            jax.ShapeDtypeStruct((n_pad, d), jnp.float32),
            jax.ShapeDtypeStruct((n_pad, d), jnp.float32),
        ],
    )(p0, p1, h, w)


def _tc_layer2_pool(p0, p1, h, w, w_out, batch2d, n_pad, t_std, t_mean):
    d = h.shape[1]
    nsteps = n_pad // _BM

    def body(p0_ref, p1_ref, h_ref, w_ref, wo_ref, b_ref, o_ref, pool_ref):
        i = pl.program_id(0)
        s = p0_ref[...] + p1_ref[...]
        h2 = (
            jnp.maximum(
                jnp.dot(
                    s,
                    w_ref[...],
                    preferred_element_type=jnp.float32,
                    precision=lax.Precision.HIGHEST,
                ),
                0.0,
            )
            + h_ref[...]
        )
        b = b_ref[...]
        gids = lax.broadcasted_iota(jnp.int32, (_G, _BM), 0)
        mask = (b == gids).astype(jnp.float32)
        contrib = jnp.dot(
            mask,
            h2,
            preferred_element_type=jnp.float32,
            precision=lax.Precision.HIGHEST,
        )

        @pl.when(i == 0)
        def _():
            pool_ref[...] = jnp.zeros((_G, d), jnp.float32)

        pool_ref[...] += contrib

        @pl.when(i == nsteps - 1)
        def _():
            g = _round_bf16(pool_ref[...])
            y = jnp.dot(
                g,
                wo_ref[...],
                preferred_element_type=jnp.float32,
                precision=lax.Precision.HIGHEST,
            )
            o_ref[...] = y[:, 0][None, :] * t_std + t_mean

    return pl.pallas_call(
        body,
        grid=(nsteps,),
        in_specs=[
            pl.BlockSpec((_BM, d), lambda i: (i, 0)),
            pl.BlockSpec((_BM, d), lambda i: (i, 0)),
            pl.BlockSpec((_BM, d), lambda i: (i, 0)),
            pl.BlockSpec((d, d), lambda i: (0, 0)),
            pl.BlockSpec((d, 1), lambda i: (0, 0)),
            pl.BlockSpec((1, _BM), lambda i: (0, i)),
        ],
        out_specs=pl.BlockSpec((1, _G), lambda i: (0, 0)),
        out_shape=jax.ShapeDtypeStruct((1, _G), jnp.float32),
        scratch_shapes=[pltpu.VMEM((_G, d), jnp.float32)],
    )(p0, p1, h, w, w_out, batch2d)


def kernel(x, edge_index, batch, W1, W2, W_out):
    n, d = x.shape
    e = edge_index.shape[1]

    n_pad = ((n + 1 + _BM - 1) // _BM) * _BM
    nch = -(-e // (_NW * _CH))
    m = _NPH * _NB
    nch = ((nch + m - 1) // m) * m
    cap = _NW * nch * _CH

    src = edge_index[0].astype(jnp.int32)
    dst = edge_index[1].astype(jnp.int32)
    pad_idx = n + (jnp.arange(cap - e, dtype=jnp.int32) % (n_pad - n))
    src_t = jnp.concatenate([src, pad_idx]).reshape(_NW, nch, _CH)
    dst_t = jnp.concatenate([dst, pad_idx]).reshape(_NW, nch, _CH)
    batch2d = jnp.concatenate(
        [batch.astype(jnp.int32), jnp.full((n_pad - n,), _G, jnp.int32)]
    ).reshape(1, n_pad)

    zpad = jnp.zeros((n_pad - n, d), jnp.float32)
    h0 = jnp.concatenate([x, zpad], axis=0)
    h0r = jnp.concatenate([_round_bf16(x), zpad], axis=0)
    w1r = _round_bf16(W1)
    w2r = _round_bf16(W2)
    wor = _round_bf16(W_out)

    p = _sc_segment_gather_add(h0r, src_t, dst_t, n_pad, nch)
    h1, h1r = _tc_layer(p[0], p[1], h0, w1r, n_pad)
    q = _sc_segment_gather_add(h1r, src_t, dst_t, n_pad, nch)
    out = _tc_layer2_pool(
        q[0], q[1], h1, w2r, wor, batch2d, n_pad, 25.0, 500.0
    )
    return out.reshape(_G)
